# baseline (speedup 1.0000x reference)
import functools

import jax
import jax.numpy as jnp
from jax import lax
from jax.experimental import pallas as pl
from jax.experimental.pallas import tpu as pltpu
from jax.experimental.pallas import tpu_sc as plsc

WORD_VOCAB = 1000000
POS_VOCAB = 50
WORD_DIM = 100
POS_DIM = 25
HIDDEN = 200
OUT = 75
B = 16384
T = 6
BT = B * T
WPAD = 128

NC, NS = 2, 16
NW = NC * NS
B_PER_W = B // NW
BCHUNK = 64
LCHUNK = BCHUNK * T
N_CHUNKS = B_PER_W // BCHUNK


def _sc_gather(word_table, widx_t):
    mesh = plsc.VectorSubcoreMesh(core_axis_name="c", subcore_axis_name="s")
    wbytes = BCHUNK * T * WPAD * 4

    @functools.partial(
        pl.kernel,
        out_type=jax.ShapeDtypeStruct((B, T * WPAD), jnp.float32),
        mesh=mesh,
        scratch_types=[
            pltpu.VMEM((T, BCHUNK), jnp.int32),
            pltpu.VMEM((T, BCHUNK), jnp.int32),
            pltpu.VMEM((BCHUNK, T * WPAD), jnp.float32),
            pltpu.VMEM((BCHUNK, T * WPAD), jnp.float32),
            pltpu.SemaphoreType.DMA,
            pltpu.SemaphoreType.DMA,
        ],
    )
    def gather_k(wt_hbm, widx_hbm, wout_hbm, idx_a, idx_b, buf_a, buf_b,
                 sem_g, sem_w):
        wid = lax.axis_index("s") * NC + lax.axis_index("c")
        b0 = wid * B_PER_W

        zeros16 = jnp.zeros((16,), jnp.float32)

        def zrow(r, carry):
            for t in range(T):
                for buf in (buf_a, buf_b):
                    buf[r, pl.ds(t * WPAD + 96, 16)] = zeros16
                    buf[r, pl.ds(t * WPAD + 112, 16)] = zeros16
            return carry

        lax.fori_loop(0, BCHUNK, zrow, 0, unroll=False)

        def fire_rows(idx_v, buf):
            copies = []
            for t in range(T):
                for g in range(BCHUNK // 16):
                    v = idx_v[t, pl.ds(g * 16, 16)]
                    for j in range(16):
                        copies.append(pltpu.async_copy(
                            wt_hbm.at[v[j]],
                            buf.at[g * 16 + j, pl.ds(t * WPAD, WORD_DIM)],
                            sem_g))
            return copies

        def wait_write():
            pltpu.make_async_copy(
                buf_b, wout_hbm.at[pl.ds(0, BCHUNK)], sem_w).wait()

        def pair_body(i, carry):
            boff_a = b0 + (2 * i) * BCHUNK
            boff_b = b0 + (2 * i + 1) * BCHUNK
            for t in range(T):
                pltpu.sync_copy(widx_hbm.at[t, pl.ds(boff_a, BCHUNK)],
                                idx_a.at[t])
            ca = fire_rows(idx_a, buf_a)

            @pl.when(i > 0)
            def _():
                wait_write()

            for cp in ca:
                cp.wait()
            pltpu.async_copy(buf_a, wout_hbm.at[pl.ds(boff_a, BCHUNK)], sem_w)

            for t in range(T):
                pltpu.sync_copy(widx_hbm.at[t, pl.ds(boff_b, BCHUNK)],
                                idx_b.at[t])
            cb = fire_rows(idx_b, buf_b)
            wait_write()
            for cp in cb:
                cp.wait()
            pltpu.async_copy(buf_b, wout_hbm.at[pl.ds(boff_b, BCHUNK)], sem_w)
            return carry

        lax.fori_loop(0, N_CHUNKS // 2, pair_body, 0, unroll=False)
        wait_write()

    return gather_k(word_table, widx_t)


def _mlp_body(xw_ref, pid_ref, a_ref, cp_ref, pt_ref, b1_ref, w2t_ref, b2_ref,
              out_ref):
    h = jnp.dot(xw_ref[...], a_ref[...], preferred_element_type=jnp.float32)
    iota50 = lax.broadcasted_iota(jnp.int32, (1, POS_VOCAB), 1)
    for t in range(T):
        p_t = jnp.dot(pt_ref[...], cp_ref[t],
                      preferred_element_type=jnp.float32)
        oh_t = (pid_ref[:, t:t + 1] == iota50).astype(jnp.float32)
        h = h + jnp.dot(oh_t, p_t, preferred_element_type=jnp.float32)
    h = jnp.maximum(h + b1_ref[...], 0.0)
    o = jnp.dot(h, w2t_ref[...], preferred_element_type=jnp.float32) + b2_ref[...]
    m = jnp.max(o, axis=1, keepdims=True)
    e = jnp.exp(o - m)
    lse = jnp.log(jnp.sum(e, axis=1, keepdims=True))
    out_ref[...] = (o - m) - lse


def _tc_mlp(xw, pos_ids, a, cp, pt, b1, w2t, b2):
    bm = 2048
    grid = (B // bm,)
    return pl.pallas_call(
        _mlp_body,
        grid=grid,
        in_specs=[
            pl.BlockSpec((bm, T * WPAD), lambda i: (i, 0)),
            pl.BlockSpec((bm, T), lambda i: (i, 0)),
            pl.BlockSpec((T * WPAD, HIDDEN), lambda i: (0, 0)),
            pl.BlockSpec((T, POS_DIM, HIDDEN), lambda i: (0, 0, 0)),
            pl.BlockSpec((POS_VOCAB, POS_DIM), lambda i: (0, 0)),
            pl.BlockSpec((1, HIDDEN), lambda i: (0, 0)),
            pl.BlockSpec((HIDDEN, OUT), lambda i: (0, 0)),
            pl.BlockSpec((1, OUT), lambda i: (0, 0)),
        ],
        out_specs=pl.BlockSpec((bm, OUT), lambda i: (i, 0)),
        out_shape=jax.ShapeDtypeStruct((B, OUT), jnp.float32),
    )(xw, pos_ids, a, cp, pt, b1, w2t, b2)


def kernel(word_ids, pos_ids, word_table, pos_table, W1, b1, W2, b2):
    widx_t = word_ids.astype(jnp.int32).T

    xw = _sc_gather(word_table, widx_t)

    w1t = W1.T.reshape(T, WORD_DIM + POS_DIM, HIDDEN)
    a = jnp.pad(w1t[:, :WORD_DIM, :],
                ((0, 0), (0, WPAD - WORD_DIM), (0, 0))).reshape(
                    T * WPAD, HIDDEN)
    cp = w1t[:, WORD_DIM:, :]

    return _tc_mlp(xw, pos_ids.astype(jnp.int32), a, cp, pos_table,
                   b1.reshape(1, HIDDEN), W2.T, b2.reshape(1, OUT))


# speedup vs baseline: 1.0076x; 1.0076x over previous
import functools

import jax
import jax.numpy as jnp
from jax import lax
from jax.experimental import pallas as pl
from jax.experimental.pallas import tpu as pltpu
from jax.experimental.pallas import tpu_sc as plsc

WORD_VOCAB = 1000000
POS_VOCAB = 50
WORD_DIM = 100
POS_DIM = 25
HIDDEN = 200
OUT = 75
B = 16384
T = 6
BT = B * T
WPAD = 128

NC, NS = 2, 16
NW = NC * NS
B_PER_W = B // NW
BCHUNK = 64
LCHUNK = BCHUNK * T
N_CHUNKS = B_PER_W // BCHUNK


def _sc_gather(word_table, widx_t):
    mesh = plsc.VectorSubcoreMesh(core_axis_name="c", subcore_axis_name="s")

    @functools.partial(
        pl.kernel,
        out_type=jax.ShapeDtypeStruct((B, T * WPAD), jnp.float32),
        mesh=mesh,
        scratch_types=[
            pltpu.VMEM((T, BCHUNK), jnp.int32),
            pltpu.VMEM((BCHUNK, T * WPAD), jnp.float32),
            pltpu.SemaphoreType.DMA,
        ],
    )
    def gather_k(wt_hbm, widx_hbm, wout_hbm, idx_v, buf_v, sem):
        wid = lax.axis_index("s") * NC + lax.axis_index("c")
        b0 = wid * B_PER_W

        zeros16 = jnp.zeros((16,), jnp.float32)

        def zrow(r, carry):
            for t in range(T):
                buf_v[r, pl.ds(t * WPAD + 96, 16)] = zeros16
                buf_v[r, pl.ds(t * WPAD + 112, 16)] = zeros16
            return carry

        lax.fori_loop(0, BCHUNK, zrow, 0, unroll=False)

        def chunk_body(c, carry):
            boff = b0 + c * BCHUNK
            for t in range(T):
                pltpu.sync_copy(widx_hbm.at[t, pl.ds(boff, BCHUNK)],
                                idx_v.at[t])
            copies = []
            for t in range(T):
                for g in range(BCHUNK // 16):
                    v = idx_v[t, pl.ds(g * 16, 16)]
                    for j in range(16):
                        copies.append(pltpu.async_copy(
                            wt_hbm.at[v[j]],
                            buf_v.at[g * 16 + j, pl.ds(t * WPAD, WORD_DIM)],
                            sem))
            for cp in copies:
                cp.wait()
            pltpu.sync_copy(buf_v, wout_hbm.at[pl.ds(boff, BCHUNK)])
            return carry

        lax.fori_loop(0, N_CHUNKS, chunk_body, 0, unroll=False)

    return gather_k(word_table, widx_t)


def _mlp_body(xw_ref, pid_ref, a_ref, cp_ref, pt_ref, b1_ref, w2t_ref, b2_ref,
              out_ref):
    h = jnp.dot(xw_ref[...], a_ref[...], preferred_element_type=jnp.float32)
    iota50 = lax.broadcasted_iota(jnp.int32, (1, POS_VOCAB), 1)
    for t in range(T):
        p_t = jnp.dot(pt_ref[...], cp_ref[t],
                      preferred_element_type=jnp.float32)
        oh_t = (pid_ref[:, t:t + 1] == iota50).astype(jnp.float32)
        h = h + jnp.dot(oh_t, p_t, preferred_element_type=jnp.float32)
    h = jnp.maximum(h + b1_ref[...], 0.0)
    o = jnp.dot(h, w2t_ref[...], preferred_element_type=jnp.float32) + b2_ref[...]
    m = jnp.max(o, axis=1, keepdims=True)
    e = jnp.exp(o - m)
    lse = jnp.log(jnp.sum(e, axis=1, keepdims=True))
    out_ref[...] = (o - m) - lse


def _tc_mlp(xw, pos_ids, a, cp, pt, b1, w2t, b2):
    bm = 2048
    grid = (B // bm,)
    return pl.pallas_call(
        _mlp_body,
        grid=grid,
        in_specs=[
            pl.BlockSpec((bm, T * WPAD), lambda i: (i, 0)),
            pl.BlockSpec((bm, T), lambda i: (i, 0)),
            pl.BlockSpec((T * WPAD, HIDDEN), lambda i: (0, 0)),
            pl.BlockSpec((T, POS_DIM, HIDDEN), lambda i: (0, 0, 0)),
            pl.BlockSpec((POS_VOCAB, POS_DIM), lambda i: (0, 0)),
            pl.BlockSpec((1, HIDDEN), lambda i: (0, 0)),
            pl.BlockSpec((HIDDEN, OUT), lambda i: (0, 0)),
            pl.BlockSpec((1, OUT), lambda i: (0, 0)),
        ],
        out_specs=pl.BlockSpec((bm, OUT), lambda i: (i, 0)),
        out_shape=jax.ShapeDtypeStruct((B, OUT), jnp.float32),
    )(xw, pos_ids, a, cp, pt, b1, w2t, b2)


def kernel(word_ids, pos_ids, word_table, pos_table, W1, b1, W2, b2):
    widx_t = word_ids.astype(jnp.int32).T

    xw = _sc_gather(word_table, widx_t)

    w1t = W1.T.reshape(T, WORD_DIM + POS_DIM, HIDDEN)
    a = jnp.pad(w1t[:, :WORD_DIM, :],
                ((0, 0), (0, WPAD - WORD_DIM), (0, 0))).reshape(
                    T * WPAD, HIDDEN)
    cp = w1t[:, WORD_DIM:, :]

    return _tc_mlp(xw, pos_ids.astype(jnp.int32), a, cp, pos_table,
                   b1.reshape(1, HIDDEN), W2.T, b2.reshape(1, OUT))


# speedup vs baseline: 1.0651x; 1.0571x over previous
import functools

import jax
import jax.numpy as jnp
from jax import lax
from jax.experimental import pallas as pl
from jax.experimental.pallas import tpu as pltpu
from jax.experimental.pallas import tpu_sc as plsc

WORD_VOCAB = 1000000
POS_VOCAB = 50
WORD_DIM = 100
POS_DIM = 25
HIDDEN = 200
OUT = 75
B = 16384
T = 6

NC, NS = 2, 16
NW = NC * NS
B_PER_W = B // NW
BCHUNK = 64
N_CHUNKS = B_PER_W // BCHUNK

PD = WORD_DIM // 2
PW = 64
PACK_LB = 4096
PACK_OB = PACK_LB // 2
N_PACK_BLKS = -(-WORD_VOCAB // PACK_LB)
PACKED_ROWS = N_PACK_BLKS * PACK_OB


def _pack_body(in_ref, out_ref):
    x = in_ref[...]
    u = lax.bitcast_convert_type(x, jnp.uint32)
    rnd = ((u >> 16) & 1) + jnp.uint32(0x7FFF)
    ub = (u + rnd) >> 16
    lo = ub[0:PD, :]
    hi = ub[PD:2 * PD, :]
    packed = lo | (hi << 16)
    pf = lax.bitcast_convert_type(packed, jnp.float32)
    pt = pf.T
    out_ref[:, 0:PD] = pt[0:PACK_OB, :]
    out_ref[:, PW:PW + PD] = pt[PACK_OB:PACK_LB, :]


def _tc_pack(wt_t):
    return pl.pallas_call(
        _pack_body,
        grid=(N_PACK_BLKS,),
        in_specs=[pl.BlockSpec((WORD_DIM, PACK_LB), lambda i: (0, i))],
        out_specs=pl.BlockSpec((PACK_OB, 2 * PW), lambda i: (i, 0)),
        out_shape=jax.ShapeDtypeStruct((PACKED_ROWS, 2 * PW), jnp.float32),
    )(wt_t)


def _sc_gather(wp, enc_t):
    mesh = plsc.VectorSubcoreMesh(core_axis_name="c", subcore_axis_name="s")

    @functools.partial(
        pl.kernel,
        out_type=jax.ShapeDtypeStruct((B, T * 2 * PW), jnp.float32),
        mesh=mesh,
        scratch_types=[
            pltpu.VMEM((T, BCHUNK), jnp.int32),
            pltpu.VMEM((BCHUNK, T * 2 * PW), jnp.float32),
            pltpu.SemaphoreType.DMA,
        ],
    )
    def gather_k(wp_hbm, enc_hbm, out_hbm, idx_v, buf_v, sem):
        wid = lax.axis_index("s") * NC + lax.axis_index("c")
        b0 = wid * B_PER_W

        def chunk_body(c, carry):
            boff = b0 + c * BCHUNK
            for t in range(T):
                pltpu.sync_copy(enc_hbm.at[t, pl.ds(boff, BCHUNK)],
                                idx_v.at[t])
            copies = []
            for t in range(T):
                for g in range(BCHUNK // 16):
                    v = idx_v[t, pl.ds(g * 16, 16)]
                    for j in range(16):
                        copies.append(pltpu.async_copy(
                            wp_hbm.at[v[j]],
                            buf_v.at[g * 16 + j, pl.ds(t * 2 * PW, 2 * PW)],
                            sem))
            for cp in copies:
                cp.wait()
            pltpu.sync_copy(buf_v, out_hbm.at[pl.ds(boff, BCHUNK)])
            return carry

        lax.fori_loop(0, N_CHUNKS, chunk_body, 0, unroll=False)

    return gather_k(wp, enc_t)


def _mlp_body(xq_ref, pid_ref, flg_ref, al_ref, ah_ref, cp_ref, pt_ref,
              b1_ref, w2t_ref, b2_ref, out_ref):
    iota128 = lax.broadcasted_iota(jnp.int32, (1, 2 * PW), 1)
    halfbit = (iota128 >> 6) & 1
    isdata = (iota128 & 63) < PD
    h = None
    for t in range(T):
        xt = xq_ref[:, t * 2 * PW:(t + 1) * 2 * PW]
        keep = (halfbit == flg_ref[:, t:t + 1]) & isdata
        xm = jnp.where(keep, xt, 0.0)
        q = lax.bitcast_convert_type(xm, jnp.uint32)
        xl = lax.bitcast_convert_type(q << 16, jnp.float32)
        xh = lax.bitcast_convert_type(q & jnp.uint32(0xFFFF0000), jnp.float32)
        ht = jnp.dot(xl, al_ref[t], preferred_element_type=jnp.float32)
        ht = ht + jnp.dot(xh, ah_ref[t], preferred_element_type=jnp.float32)
        h = ht if h is None else h + ht
    iota50 = lax.broadcasted_iota(jnp.int32, (1, POS_VOCAB), 1)
    for t in range(T):
        p_t = jnp.dot(pt_ref[...], cp_ref[t],
                      preferred_element_type=jnp.float32)
        oh_t = (pid_ref[:, t:t + 1] == iota50).astype(jnp.float32)
        h = h + jnp.dot(oh_t, p_t, preferred_element_type=jnp.float32)
    h = jnp.maximum(h + b1_ref[...], 0.0)
    o = jnp.dot(h, w2t_ref[...], preferred_element_type=jnp.float32) + b2_ref[...]
    m = jnp.max(o, axis=1, keepdims=True)
    e = jnp.exp(o - m)
    lse = jnp.log(jnp.sum(e, axis=1, keepdims=True))
    out_ref[...] = (o - m) - lse


def _tc_mlp(xq, pos_ids, flags, al, ah, cp, pt, b1, w2t, b2):
    bm = 2048
    grid = (B // bm,)
    return pl.pallas_call(
        _mlp_body,
        grid=grid,
        in_specs=[
            pl.BlockSpec((bm, T * 2 * PW), lambda i: (i, 0)),
            pl.BlockSpec((bm, T), lambda i: (i, 0)),
            pl.BlockSpec((bm, T), lambda i: (i, 0)),
            pl.BlockSpec((T, 2 * PW, HIDDEN), lambda i: (0, 0, 0)),
            pl.BlockSpec((T, 2 * PW, HIDDEN), lambda i: (0, 0, 0)),
            pl.BlockSpec((T, POS_DIM, HIDDEN), lambda i: (0, 0, 0)),
            pl.BlockSpec((POS_VOCAB, POS_DIM), lambda i: (0, 0)),
            pl.BlockSpec((1, HIDDEN), lambda i: (0, 0)),
            pl.BlockSpec((HIDDEN, OUT), lambda i: (0, 0)),
            pl.BlockSpec((1, OUT), lambda i: (0, 0)),
        ],
        out_specs=pl.BlockSpec((bm, OUT), lambda i: (i, 0)),
        out_shape=jax.ShapeDtypeStruct((B, OUT), jnp.float32),
    )(xq, pos_ids, flags, al, ah, cp, pt, b1, w2t, b2)


def kernel(word_ids, pos_ids, word_table, pos_table, W1, b1, W2, b2):
    wp = _tc_pack(word_table.T)

    widx_t = word_ids.astype(jnp.int32).T
    enc_t = ((widx_t >> 12) << 11) | (widx_t & 2047)
    flags = ((word_ids.astype(jnp.int32) >> 11) & 1)

    xq = _sc_gather(wp, enc_t)

    w1t = W1.T.reshape(T, WORD_DIM + POS_DIM, HIDDEN)
    alq = jnp.pad(w1t[:, :PD, :], ((0, 0), (0, PW - PD), (0, 0)))
    ahq = jnp.pad(w1t[:, PD:2 * PD, :], ((0, 0), (0, PW - PD), (0, 0)))
    al = jnp.concatenate([alq, alq], axis=1)
    ah = jnp.concatenate([ahq, ahq], axis=1)
    cp = w1t[:, WORD_DIM:, :]

    return _tc_mlp(xq, pos_ids.astype(jnp.int32), flags, al, ah, cp,
                   pos_table, b1.reshape(1, HIDDEN), W2.T,
                   b2.reshape(1, OUT))


# speedup vs baseline: 1.2332x; 1.1578x over previous
import functools

import jax
import jax.numpy as jnp
from jax import lax
from jax.experimental import pallas as pl
from jax.experimental.pallas import tpu as pltpu
from jax.experimental.pallas import tpu_sc as plsc

WORD_VOCAB = 1000000
POS_VOCAB = 50
WORD_DIM = 100
POS_DIM = 25
HIDDEN = 200
OUT = 75
B = 16384
T = 6

NC, NS = 2, 16
NW = NC * NS
B_PER_W = B // NW
BCHUNK = 64
N_CHUNKS = B_PER_W // BCHUNK

PD = WORD_DIM // 2
PW = 64
PACK_LB = 8192
PACK_OB = PACK_LB // 2
PACK_LBITS = PACK_LB.bit_length() - 1
N_PACK_BLKS = -(-WORD_VOCAB // PACK_LB)
PACKED_ROWS = N_PACK_BLKS * PACK_OB


def _pack_body(in_ref, out_ref):
    x = in_ref[...]
    u = lax.bitcast_convert_type(x, jnp.uint32)
    rnd = ((u >> 16) & 1) + jnp.uint32(0x7FFF)
    ub = (u + rnd) >> 16
    lo = ub[0:PD, :]
    hi = ub[PD:2 * PD, :]
    packed = lo | (hi << 16)
    pf = lax.bitcast_convert_type(packed, jnp.float32)
    pt = pf.T
    out_ref[:, 0:PD] = pt[0:PACK_OB, :]
    out_ref[:, PW:PW + PD] = pt[PACK_OB:PACK_LB, :]


def _tc_pack(wt_t):
    return pl.pallas_call(
        _pack_body,
        grid=(N_PACK_BLKS,),
        in_specs=[pl.BlockSpec((WORD_DIM, PACK_LB), lambda i: (0, i))],
        out_specs=pl.BlockSpec((PACK_OB, 2 * PW), lambda i: (i, 0)),
        out_shape=jax.ShapeDtypeStruct((PACKED_ROWS, 2 * PW), jnp.float32),
    )(wt_t)


def _sc_gather(wp, enc_t):
    mesh = plsc.VectorSubcoreMesh(core_axis_name="c", subcore_axis_name="s")

    @functools.partial(
        pl.kernel,
        out_type=jax.ShapeDtypeStruct((B, T * 2 * PW), jnp.float32),
        mesh=mesh,
        scratch_types=[
            pltpu.VMEM((T, BCHUNK), jnp.int32),
            pltpu.VMEM((BCHUNK, T * 2 * PW), jnp.float32),
            pltpu.SemaphoreType.DMA,
        ],
    )
    def gather_k(wp_hbm, enc_hbm, out_hbm, idx_v, buf_v, sem):
        wid = lax.axis_index("s") * NC + lax.axis_index("c")
        b0 = wid * B_PER_W

        def chunk_body(c, carry):
            boff = b0 + c * BCHUNK
            for t in range(T):
                pltpu.sync_copy(enc_hbm.at[t, pl.ds(boff, BCHUNK)],
                                idx_v.at[t])
            copies = []
            for t in range(T):
                for g in range(BCHUNK // 16):
                    v = idx_v[t, pl.ds(g * 16, 16)]
                    for j in range(16):
                        copies.append(pltpu.async_copy(
                            wp_hbm.at[v[j]],
                            buf_v.at[g * 16 + j, pl.ds(t * 2 * PW, 2 * PW)],
                            sem))
            for cp in copies:
                cp.wait()
            pltpu.sync_copy(buf_v, out_hbm.at[pl.ds(boff, BCHUNK)])
            return carry

        lax.fori_loop(0, N_CHUNKS, chunk_body, 0, unroll=False)

    return gather_k(wp, enc_t)


def _mlp_body(xq_ref, pid_ref, flg_ref, al_ref, ah_ref, cp_ref, pt_ref,
              b1_ref, w2t_ref, b2_ref, out_ref):
    iota128 = lax.broadcasted_iota(jnp.int32, (1, 2 * PW), 1)
    halfbit = (iota128 >> 6) & 1
    isdata = (iota128 & 63) < PD
    h = None
    for t in range(T):
        xt = xq_ref[:, t * 2 * PW:(t + 1) * 2 * PW]
        keep = (halfbit == flg_ref[:, t:t + 1]) & isdata
        xm = jnp.where(keep, xt, 0.0)
        q = lax.bitcast_convert_type(xm, jnp.uint32)
        xl = lax.bitcast_convert_type(q << 16, jnp.float32)
        xh = lax.bitcast_convert_type(q & jnp.uint32(0xFFFF0000), jnp.float32)
        ht = jnp.dot(xl, al_ref[t], preferred_element_type=jnp.float32)
        ht = ht + jnp.dot(xh, ah_ref[t], preferred_element_type=jnp.float32)
        h = ht if h is None else h + ht
    iota50 = lax.broadcasted_iota(jnp.int32, (1, POS_VOCAB), 1)
    for t in range(T):
        p_t = jnp.dot(pt_ref[...], cp_ref[t],
                      preferred_element_type=jnp.float32)
        oh_t = (pid_ref[:, t:t + 1] == iota50).astype(jnp.float32)
        h = h + jnp.dot(oh_t, p_t, preferred_element_type=jnp.float32)
    h = jnp.maximum(h + b1_ref[...], 0.0)
    o = jnp.dot(h, w2t_ref[...], preferred_element_type=jnp.float32) + b2_ref[...]
    m = jnp.max(o, axis=1, keepdims=True)
    e = jnp.exp(o - m)
    lse = jnp.log(jnp.sum(e, axis=1, keepdims=True))
    out_ref[...] = (o - m) - lse


def _tc_mlp(xq, pos_ids, flags, al, ah, cp, pt, b1, w2t, b2):
    bm = 4096
    grid = (B // bm,)
    return pl.pallas_call(
        _mlp_body,
        grid=grid,
        in_specs=[
            pl.BlockSpec((bm, T * 2 * PW), lambda i: (i, 0)),
            pl.BlockSpec((bm, T), lambda i: (i, 0)),
            pl.BlockSpec((bm, T), lambda i: (i, 0)),
            pl.BlockSpec((T, 2 * PW, HIDDEN), lambda i: (0, 0, 0)),
            pl.BlockSpec((T, 2 * PW, HIDDEN), lambda i: (0, 0, 0)),
            pl.BlockSpec((T, POS_DIM, HIDDEN), lambda i: (0, 0, 0)),
            pl.BlockSpec((POS_VOCAB, POS_DIM), lambda i: (0, 0)),
            pl.BlockSpec((1, HIDDEN), lambda i: (0, 0)),
            pl.BlockSpec((HIDDEN, OUT), lambda i: (0, 0)),
            pl.BlockSpec((1, OUT), lambda i: (0, 0)),
        ],
        out_specs=pl.BlockSpec((bm, OUT), lambda i: (i, 0)),
        out_shape=jax.ShapeDtypeStruct((B, OUT), jnp.float32),
    )(xq, pos_ids, flags, al, ah, cp, pt, b1, w2t, b2)


def kernel(word_ids, pos_ids, word_table, pos_table, W1, b1, W2, b2):
    wp = _tc_pack(word_table.T)

    widx_t = word_ids.astype(jnp.int32).T
    enc_t = ((widx_t >> PACK_LBITS) << (PACK_LBITS - 1)) \
        | (widx_t & (PACK_OB - 1))
    flags = ((word_ids.astype(jnp.int32) >> (PACK_LBITS - 1)) & 1)

    xq = _sc_gather(wp, enc_t)

    w1t = W1.T.reshape(T, WORD_DIM + POS_DIM, HIDDEN)
    alq = jnp.pad(w1t[:, :PD, :], ((0, 0), (0, PW - PD), (0, 0)))
    ahq = jnp.pad(w1t[:, PD:2 * PD, :], ((0, 0), (0, PW - PD), (0, 0)))
    al = jnp.concatenate([alq, alq], axis=1)
    ah = jnp.concatenate([ahq, ahq], axis=1)
    cp = w1t[:, WORD_DIM:, :]

    return _tc_mlp(xq, pos_ids.astype(jnp.int32), flags, al, ah, cp,
                   pos_table, b1.reshape(1, HIDDEN), W2.T,
                   b2.reshape(1, OUT))


# speedup vs baseline: 1.3416x; 1.0879x over previous
import functools

import jax
import jax.numpy as jnp
from jax import lax
from jax.experimental import pallas as pl
from jax.experimental.pallas import tpu as pltpu
from jax.experimental.pallas import tpu_sc as plsc

WORD_VOCAB = 1000000
POS_VOCAB = 50
WORD_DIM = 100
POS_DIM = 25
HIDDEN = 200
OUT = 75
B = 16384
T = 6

NC, NS = 2, 16
NW = NC * NS
B_PER_W = B // NW
BCHUNK = 64
N_CHUNKS = B_PER_W // BCHUNK

PD = WORD_DIM // 2
PW = 64
PACK_LB = 16384
PACK_OB = PACK_LB // 2
PACK_LBITS = PACK_LB.bit_length() - 1
N_PACK_BLKS = -(-WORD_VOCAB // PACK_LB)
PACKED_ROWS = N_PACK_BLKS * PACK_OB


def _pack_body(in_ref, out_ref):
    x = in_ref[...]
    u = lax.bitcast_convert_type(x, jnp.uint32)
    rnd = ((u >> 16) & 1) + jnp.uint32(0x7FFF)
    ub = (u + rnd) >> 16
    lo = ub[0:PD, :]
    hi = ub[PD:2 * PD, :]
    packed = lo | (hi << 16)
    pf = lax.bitcast_convert_type(packed, jnp.float32)
    pt = pf.T
    out_ref[:, 0:PD] = pt[0:PACK_OB, :]
    out_ref[:, PW:PW + PD] = pt[PACK_OB:PACK_LB, :]


def _tc_pack(wt_t):
    return pl.pallas_call(
        _pack_body,
        grid=(N_PACK_BLKS,),
        in_specs=[pl.BlockSpec((WORD_DIM, PACK_LB), lambda i: (0, i))],
        out_specs=pl.BlockSpec((PACK_OB, 2 * PW), lambda i: (i, 0)),
        out_shape=jax.ShapeDtypeStruct((PACKED_ROWS, 2 * PW), jnp.float32),
    )(wt_t)


def _sc_gather(wp, enc_t):
    mesh = plsc.VectorSubcoreMesh(core_axis_name="c", subcore_axis_name="s")

    @functools.partial(
        pl.kernel,
        out_type=jax.ShapeDtypeStruct((B, T * 2 * PW), jnp.float32),
        mesh=mesh,
        scratch_types=[
            pltpu.VMEM((T, BCHUNK), jnp.int32),
            pltpu.VMEM((BCHUNK, T * 2 * PW), jnp.float32),
            pltpu.SemaphoreType.DMA,
        ],
    )
    def gather_k(wp_hbm, enc_hbm, out_hbm, idx_v, buf_v, sem):
        wid = lax.axis_index("s") * NC + lax.axis_index("c")
        b0 = wid * B_PER_W

        def chunk_body(c, carry):
            boff = b0 + c * BCHUNK
            for t in range(T):
                pltpu.sync_copy(enc_hbm.at[t, pl.ds(boff, BCHUNK)],
                                idx_v.at[t])
            copies = []
            for t in range(T):
                for g in range(BCHUNK // 16):
                    v = idx_v[t, pl.ds(g * 16, 16)]
                    for j in range(16):
                        copies.append(pltpu.async_copy(
                            wp_hbm.at[v[j]],
                            buf_v.at[g * 16 + j, pl.ds(t * 2 * PW, 2 * PW)],
                            sem))
            for cp in copies:
                cp.wait()
            pltpu.sync_copy(buf_v, out_hbm.at[pl.ds(boff, BCHUNK)])
            return carry

        lax.fori_loop(0, N_CHUNKS, chunk_body, 0, unroll=False)

    return gather_k(wp, enc_t)


def _mlp_body(xq_ref, pid_ref, flg_ref, al_ref, ah_ref, cp_ref, pt_ref,
              b1_ref, w2t_ref, b2_ref, out_ref):
    iota128 = lax.broadcasted_iota(jnp.int32, (1, 2 * PW), 1)
    halfbit = (iota128 >> 6) & 1
    isdata = (iota128 & 63) < PD
    h = None
    for t in range(T):
        xt = xq_ref[:, t * 2 * PW:(t + 1) * 2 * PW]
        keep = (halfbit == flg_ref[:, t:t + 1]) & isdata
        xm = jnp.where(keep, xt, 0.0)
        q = lax.bitcast_convert_type(xm, jnp.uint32)
        xl = lax.bitcast_convert_type(q << 16, jnp.float32)
        xh = lax.bitcast_convert_type(q & jnp.uint32(0xFFFF0000), jnp.float32)
        ht = jnp.dot(xl, al_ref[t], preferred_element_type=jnp.float32)
        ht = ht + jnp.dot(xh, ah_ref[t], preferred_element_type=jnp.float32)
        h = ht if h is None else h + ht
    iota50 = lax.broadcasted_iota(jnp.int32, (1, POS_VOCAB), 1)
    for t in range(T):
        p_t = jnp.dot(pt_ref[...], cp_ref[t],
                      preferred_element_type=jnp.float32)
        oh_t = (pid_ref[:, t:t + 1] == iota50).astype(jnp.float32)
        h = h + jnp.dot(oh_t, p_t, preferred_element_type=jnp.float32)
    h = jnp.maximum(h + b1_ref[...], 0.0)
    o = jnp.dot(h, w2t_ref[...], preferred_element_type=jnp.float32) + b2_ref[...]
    m = jnp.max(o, axis=1, keepdims=True)
    e = jnp.exp(o - m)
    lse = jnp.log(jnp.sum(e, axis=1, keepdims=True))
    out_ref[...] = (o - m) - lse


def _tc_mlp(xq, pos_ids, flags, al, ah, cp, pt, b1, w2t, b2):
    bm = 4096
    grid = (B // bm,)
    return pl.pallas_call(
        _mlp_body,
        grid=grid,
        in_specs=[
            pl.BlockSpec((bm, T * 2 * PW), lambda i: (i, 0)),
            pl.BlockSpec((bm, T), lambda i: (i, 0)),
            pl.BlockSpec((bm, T), lambda i: (i, 0)),
            pl.BlockSpec((T, 2 * PW, HIDDEN), lambda i: (0, 0, 0)),
            pl.BlockSpec((T, 2 * PW, HIDDEN), lambda i: (0, 0, 0)),
            pl.BlockSpec((T, POS_DIM, HIDDEN), lambda i: (0, 0, 0)),
            pl.BlockSpec((POS_VOCAB, POS_DIM), lambda i: (0, 0)),
            pl.BlockSpec((1, HIDDEN), lambda i: (0, 0)),
            pl.BlockSpec((HIDDEN, OUT), lambda i: (0, 0)),
            pl.BlockSpec((1, OUT), lambda i: (0, 0)),
        ],
        out_specs=pl.BlockSpec((bm, OUT), lambda i: (i, 0)),
        out_shape=jax.ShapeDtypeStruct((B, OUT), jnp.float32),
    )(xq, pos_ids, flags, al, ah, cp, pt, b1, w2t, b2)


def kernel(word_ids, pos_ids, word_table, pos_table, W1, b1, W2, b2):
    wp = _tc_pack(word_table.T)

    widx_t = word_ids.astype(jnp.int32).T
    enc_t = ((widx_t >> PACK_LBITS) << (PACK_LBITS - 1)) \
        | (widx_t & (PACK_OB - 1))
    flags = ((word_ids.astype(jnp.int32) >> (PACK_LBITS - 1)) & 1)

    xq = _sc_gather(wp, enc_t)

    w1t = W1.T.reshape(T, WORD_DIM + POS_DIM, HIDDEN)
    alq = jnp.pad(w1t[:, :PD, :], ((0, 0), (0, PW - PD), (0, 0)))
    ahq = jnp.pad(w1t[:, PD:2 * PD, :], ((0, 0), (0, PW - PD), (0, 0)))
    al = jnp.concatenate([alq, alq], axis=1)
    ah = jnp.concatenate([ahq, ahq], axis=1)
    cp = w1t[:, WORD_DIM:, :]

    return _tc_mlp(xq, pos_ids.astype(jnp.int32), flags, al, ah, cp,
                   pos_table, b1.reshape(1, HIDDEN), W2.T,
                   b2.reshape(1, OUT))


# speedup vs baseline: 1.3999x; 1.0435x over previous
import functools

import jax
import jax.numpy as jnp
from jax import lax
from jax.experimental import pallas as pl
from jax.experimental.pallas import tpu as pltpu
from jax.experimental.pallas import tpu_sc as plsc

WORD_VOCAB = 1000000
POS_VOCAB = 50
WORD_DIM = 100
POS_DIM = 25
HIDDEN = 200
OUT = 75
B = 16384
T = 6

NC, NS = 2, 16
NW = NC * NS
B_PER_W = B // NW
BCHUNK = 64
N_CHUNKS = B_PER_W // BCHUNK

PD = WORD_DIM // 2
PW = 64
PACK_LB = 32768
PACK_OB = PACK_LB // 2
PACK_LBITS = PACK_LB.bit_length() - 1
N_PACK_BLKS = -(-WORD_VOCAB // PACK_LB)
PACKED_ROWS = N_PACK_BLKS * PACK_OB


def _pack_body(in_ref, out_ref):
    x = in_ref[...]
    u = lax.bitcast_convert_type(x, jnp.uint32)
    rnd = ((u >> 16) & 1) + jnp.uint32(0x7FFF)
    ub = (u + rnd) >> 16
    lo = ub[0:PD, :]
    hi = ub[PD:2 * PD, :]
    packed = lo | (hi << 16)
    pf = lax.bitcast_convert_type(packed, jnp.float32)
    pt = pf.T
    out_ref[:, 0:PD] = pt[0:PACK_OB, :]
    out_ref[:, PW:PW + PD] = pt[PACK_OB:PACK_LB, :]


def _tc_pack(wt_t):
    return pl.pallas_call(
        _pack_body,
        grid=(N_PACK_BLKS,),
        in_specs=[pl.BlockSpec((WORD_DIM, PACK_LB), lambda i: (0, i))],
        out_specs=pl.BlockSpec((PACK_OB, 2 * PW), lambda i: (i, 0)),
        out_shape=jax.ShapeDtypeStruct((PACKED_ROWS, 2 * PW), jnp.float32),
    )(wt_t)


def _sc_gather(wp, enc_t):
    mesh = plsc.VectorSubcoreMesh(core_axis_name="c", subcore_axis_name="s")

    @functools.partial(
        pl.kernel,
        out_type=jax.ShapeDtypeStruct((B, T * 2 * PW), jnp.float32),
        mesh=mesh,
        scratch_types=[
            pltpu.VMEM((T, BCHUNK), jnp.int32),
            pltpu.VMEM((BCHUNK, T * 2 * PW), jnp.float32),
            pltpu.SemaphoreType.DMA,
        ],
    )
    def gather_k(wp_hbm, enc_hbm, out_hbm, idx_v, buf_v, sem):
        wid = lax.axis_index("s") * NC + lax.axis_index("c")
        b0 = wid * B_PER_W

        def chunk_body(c, carry):
            boff = b0 + c * BCHUNK
            for t in range(T):
                pltpu.sync_copy(enc_hbm.at[t, pl.ds(boff, BCHUNK)],
                                idx_v.at[t])
            copies = []
            for t in range(T):
                for g in range(BCHUNK // 16):
                    v = idx_v[t, pl.ds(g * 16, 16)]
                    for j in range(16):
                        copies.append(pltpu.async_copy(
                            wp_hbm.at[v[j]],
                            buf_v.at[g * 16 + j, pl.ds(t * 2 * PW, 2 * PW)],
                            sem))
            for cp in copies:
                cp.wait()
            pltpu.sync_copy(buf_v, out_hbm.at[pl.ds(boff, BCHUNK)])
            return carry

        lax.fori_loop(0, N_CHUNKS, chunk_body, 0, unroll=False)

    return gather_k(wp, enc_t)


def _mlp_body(xq_ref, pid_ref, flg_ref, al_ref, ah_ref, cp_ref, pt_ref,
              b1_ref, w2t_ref, b2_ref, out_ref):
    iota128 = lax.broadcasted_iota(jnp.int32, (1, 2 * PW), 1)
    halfbit = (iota128 >> 6) & 1
    isdata = (iota128 & 63) < PD
    h = None
    for t in range(T):
        xt = xq_ref[:, t * 2 * PW:(t + 1) * 2 * PW]
        keep = (halfbit == flg_ref[:, t:t + 1]) & isdata
        xm = jnp.where(keep, xt, 0.0)
        q = lax.bitcast_convert_type(xm, jnp.uint32)
        xl = lax.bitcast_convert_type(q << 16, jnp.float32)
        xh = lax.bitcast_convert_type(q & jnp.uint32(0xFFFF0000), jnp.float32)
        ht = jnp.dot(xl, al_ref[t], preferred_element_type=jnp.float32)
        ht = ht + jnp.dot(xh, ah_ref[t], preferred_element_type=jnp.float32)
        h = ht if h is None else h + ht
    iota50 = lax.broadcasted_iota(jnp.int32, (1, POS_VOCAB), 1)
    for t in range(T):
        p_t = jnp.dot(pt_ref[...], cp_ref[t],
                      preferred_element_type=jnp.float32)
        oh_t = (pid_ref[:, t:t + 1] == iota50).astype(jnp.float32)
        h = h + jnp.dot(oh_t, p_t, preferred_element_type=jnp.float32)
    h = jnp.maximum(h + b1_ref[...], 0.0)
    o = jnp.dot(h, w2t_ref[...], preferred_element_type=jnp.float32) + b2_ref[...]
    m = jnp.max(o, axis=1, keepdims=True)
    e = jnp.exp(o - m)
    lse = jnp.log(jnp.sum(e, axis=1, keepdims=True))
    out_ref[...] = (o - m) - lse


def _tc_mlp(xq, pos_ids, flags, al, ah, cp, pt, b1, w2t, b2):
    bm = 4096
    grid = (B // bm,)
    return pl.pallas_call(
        _mlp_body,
        grid=grid,
        in_specs=[
            pl.BlockSpec((bm, T * 2 * PW), lambda i: (i, 0)),
            pl.BlockSpec((bm, T), lambda i: (i, 0)),
            pl.BlockSpec((bm, T), lambda i: (i, 0)),
            pl.BlockSpec((T, 2 * PW, HIDDEN), lambda i: (0, 0, 0)),
            pl.BlockSpec((T, 2 * PW, HIDDEN), lambda i: (0, 0, 0)),
            pl.BlockSpec((T, POS_DIM, HIDDEN), lambda i: (0, 0, 0)),
            pl.BlockSpec((POS_VOCAB, POS_DIM), lambda i: (0, 0)),
            pl.BlockSpec((1, HIDDEN), lambda i: (0, 0)),
            pl.BlockSpec((HIDDEN, OUT), lambda i: (0, 0)),
            pl.BlockSpec((1, OUT), lambda i: (0, 0)),
        ],
        out_specs=pl.BlockSpec((bm, OUT), lambda i: (i, 0)),
        out_shape=jax.ShapeDtypeStruct((B, OUT), jnp.float32),
    )(xq, pos_ids, flags, al, ah, cp, pt, b1, w2t, b2)


def kernel(word_ids, pos_ids, word_table, pos_table, W1, b1, W2, b2):
    wp = _tc_pack(word_table.T)

    widx_t = word_ids.astype(jnp.int32).T
    enc_t = ((widx_t >> PACK_LBITS) << (PACK_LBITS - 1)) \
        | (widx_t & (PACK_OB - 1))
    flags = ((word_ids.astype(jnp.int32) >> (PACK_LBITS - 1)) & 1)

    xq = _sc_gather(wp, enc_t)

    w1t = W1.T.reshape(T, WORD_DIM + POS_DIM, HIDDEN)
    alq = jnp.pad(w1t[:, :PD, :], ((0, 0), (0, PW - PD), (0, 0)))
    ahq = jnp.pad(w1t[:, PD:2 * PD, :], ((0, 0), (0, PW - PD), (0, 0)))
    al = jnp.concatenate([alq, alq], axis=1)
    ah = jnp.concatenate([ahq, ahq], axis=1)
    cp = w1t[:, WORD_DIM:, :]

    return _tc_mlp(xq, pos_ids.astype(jnp.int32), flags, al, ah, cp,
                   pos_table, b1.reshape(1, HIDDEN), W2.T,
                   b2.reshape(1, OUT))
